# Initial kernel scaffold; baseline (speedup 1.0000x reference)
import jax
import jax.numpy as jnp
from jax.experimental import pallas as pl


def kernel(x, edge_index, coord, edge_attr, W_e1, b_e1, W_e2, b_e2, W_n1, b_n1, W_n2, b_n2, W_c1, b_c1, W_c2):
    raise NotImplementedError("write your pallas kernel here")


# speedup vs baseline: 3.0702x; 3.0702x over previous
import functools

import jax
import jax.numpy as jnp
from jax import lax
from jax.experimental import pallas as pl
from jax.experimental.pallas import tpu as pltpu
from jax.experimental.pallas import tpu_sc as plsc

N = 10000
E = 320000
D = 128
H = 128
DE = 16

NC = 2
NS = 16
NW = NC * NS
EPW = E // NW
CHUNK = 80
NCHUNK = EPW // CHUNK
N_TAB = 10240
ROWS_PT = N_TAB // NS
ZCH = 128

_f32 = jnp.float32
_DEBUG_JNP_GATHER = False
_DEBUG_JNP_SCATTER = False
_DEBUG_JNP_SCATTER_SMALL = True


def _silu(v):
    return v * (1.0 / (1.0 + jnp.exp(-v)))


def _pq_body(x_ref, a_ref, b_ref, p_ref, q_ref):
    xb = x_ref[...]
    p_ref[...] = jnp.dot(xb, a_ref[...], preferred_element_type=_f32)
    q_ref[...] = jnp.dot(xb, b_ref[...], preferred_element_type=_f32)


def _compute_pq(x, A, B):
    blk = 1000
    return pl.pallas_call(
        _pq_body,
        grid=(N // blk,),
        in_specs=[
            pl.BlockSpec((blk, D), lambda i: (i, 0)),
            pl.BlockSpec((D, H), lambda i: (0, 0)),
            pl.BlockSpec((D, H), lambda i: (0, 0)),
        ],
        out_specs=[pl.BlockSpec((blk, H), lambda i: (i, 0))] * 2,
        out_shape=[jax.ShapeDtypeStruct((N, H), _f32)] * 2,
    )(x, A, B)


def _gather_body(p_hbm, q_hbm, crd_hbm, row_hbm, col_hbm,
                 gp_out, gq_out, cd_out,
                 rowv, colv, pg, qg, cdb, crd_v, sem):
    pltpu.sync_copy(crd_hbm, crd_v)

    wid = lax.axis_index("s") * NC + lax.axis_index("c")
    base = wid * EPW

    def chunk_body(i, carry):
        off = base + i * CHUNK
        pltpu.sync_copy(row_hbm.at[pl.ds(off, CHUNK)], rowv)
        pltpu.sync_copy(col_hbm.at[pl.ds(off, CHUNK)], colv)
        c1 = pltpu.async_copy(p_hbm.at[rowv], pg, sem)
        c2 = pltpu.async_copy(q_hbm.at[colv], qg, sem)
        for g in range(CHUNK // 16):
            rid = rowv[pl.ds(g * 16, 16)] * 4
            cid2 = colv[pl.ds(g * 16, 16)] * 4
            eidx = jnp.full((16,), g * 16, jnp.int32) + lax.iota(jnp.int32, 16)
            acc = jnp.zeros((16,), _f32)
            for k in range(3):
                a = plsc.load_gather(crd_v, [rid + k])
                b = plsc.load_gather(crd_v, [cid2 + k])
                dk = a - b
                acc = acc + dk * dk
                plsc.store_scatter(cdb, [eidx, jnp.full((16,), k, jnp.int32)],
                                   dk)
            plsc.store_scatter(cdb, [eidx, jnp.full((16,), 3, jnp.int32)],
                               acc)
        c1.wait()
        c2.wait()
        pltpu.sync_copy(pg, gp_out.at[pl.ds(off, CHUNK)])
        pltpu.sync_copy(qg, gq_out.at[pl.ds(off, CHUNK)])
        pltpu.sync_copy(cdb, cd_out.at[pl.ds(off, CHUNK)])
        return carry

    lax.fori_loop(0, NCHUNK, chunk_body, 0)


def _gather(P, Q, crd_flat, row, col):
    mesh = plsc.VectorSubcoreMesh(core_axis_name="c", subcore_axis_name="s")
    f = functools.partial(
        pl.kernel,
        out_type=(
            jax.ShapeDtypeStruct((E, H), _f32),
            jax.ShapeDtypeStruct((E, H), _f32),
            jax.ShapeDtypeStruct((E, 4), _f32),
        ),
        mesh=mesh,
        compiler_params=pltpu.CompilerParams(needs_layout_passes=False),
        scratch_types=[
            pltpu.VMEM((CHUNK,), jnp.int32),
            pltpu.VMEM((CHUNK,), jnp.int32),
            pltpu.VMEM((CHUNK, H), _f32),
            pltpu.VMEM((CHUNK, H), _f32),
            pltpu.VMEM((CHUNK, 4), _f32),
            pltpu.VMEM((4 * N,), _f32),
            pltpu.SemaphoreType.DMA,
        ],
    )(_gather_body)
    return f(P, Q, crd_flat, row, col)


def _edge_body(gp, gq, cd_ref, ea, wr, cw, be1, we2, be2, wc1, bc1, wc2,
               ef_o, sm_o):
    g = gp[...] + gq[...]
    cd = cd_ref[...]
    radial = cd[:, 3:4]
    pre = (g + radial * wr[...]
           + jnp.dot(ea[...], cw[...], preferred_element_type=_f32)
           + be1[...])
    h = _silu(pre)
    ef = _silu(jnp.dot(h, we2[...], preferred_element_type=_f32) + be2[...])
    ch = _silu(jnp.dot(ef, wc1[...], preferred_element_type=_f32) + bc1[...])
    s = jnp.dot(ch, wc2[...], preferred_element_type=_f32)
    ef_o[...] = ef
    blk = cd.shape[0]
    sm_o[...] = jnp.concatenate(
        [cd * s, jnp.ones((blk, 1), _f32), jnp.zeros((blk, 11), _f32)],
        axis=1)


def _edge_mlp(gp, gq, cd, edge_attr, wr, cw, be1, we2, be2, wc1, bc1, wc2):
    BE = 2000
    full = lambda i: (0, 0)
    return pl.pallas_call(
        _edge_body,
        grid=(E // BE,),
        in_specs=[
            pl.BlockSpec((BE, H), lambda i: (i, 0)),
            pl.BlockSpec((BE, H), lambda i: (i, 0)),
            pl.BlockSpec((BE, 4), lambda i: (i, 0)),
            pl.BlockSpec((BE, DE), lambda i: (i, 0)),
            pl.BlockSpec((1, H), full),
            pl.BlockSpec((DE, H), full),
            pl.BlockSpec((1, H), full),
            pl.BlockSpec((H, H), full),
            pl.BlockSpec((1, H), full),
            pl.BlockSpec((H, H), full),
            pl.BlockSpec((1, H), full),
            pl.BlockSpec((H, 1), full),
        ],
        out_specs=[
            pl.BlockSpec((BE, H), lambda i: (i, 0)),
            pl.BlockSpec((BE, 16), lambda i: (i, 0)),
        ],
        out_shape=[
            jax.ShapeDtypeStruct((E, H), _f32),
            jax.ShapeDtypeStruct((E, 16), _f32),
        ],
    )(gp, gq, cd, edge_attr, wr, cw, be1, we2, be2, wc1, bc1, wc2)


def _scatter_big_body(ef_hbm, row_hbm, big_out, rowv, efv, zb, table, sem):
    cid = lax.axis_index("c")
    sid = lax.axis_index("s")
    wid = sid * NC + cid

    def zero_zb(e, carry):
        for u in range(H // 16):
            zb[e, pl.ds(u * 16, 16)] = jnp.zeros((16,), _f32)
        return carry

    lax.fori_loop(0, ZCH, zero_zb, 0)
    for r in range(ROWS_PT // ZCH):
        start = sid * ROWS_PT + r * ZCH
        pltpu.sync_copy(zb, table.at[pl.ds(start, ZCH)])
    plsc.subcore_barrier()

    base = wid * EPW

    def chunk_body(i, carry):
        off = base + i * CHUNK
        pltpu.sync_copy(row_hbm.at[pl.ds(off, CHUNK)], rowv)
        pltpu.sync_copy(ef_hbm.at[pl.ds(off, CHUNK)], efv)
        pltpu.sync_copy(efv, table.at[rowv], add=True)
        return carry

    lax.fori_loop(0, NCHUNK, chunk_body, 0)
    plsc.subcore_barrier()

    start = sid * ROWS_PT
    pltpu.sync_copy(table.at[pl.ds(start, ROWS_PT)],
                    big_out.at[cid, pl.ds(start, ROWS_PT)])


def _scatter_big(ef, row):
    mesh = plsc.VectorSubcoreMesh(core_axis_name="c", subcore_axis_name="s")
    f = functools.partial(
        pl.kernel,
        out_type=jax.ShapeDtypeStruct((NC, N_TAB, H), _f32),
        mesh=mesh,
        scratch_types=[
            pltpu.VMEM((CHUNK,), jnp.int32),
            pltpu.VMEM((CHUNK, H), _f32),
            pltpu.VMEM((ZCH, H), _f32),
            pltpu.VMEM_SHARED((N_TAB, H), _f32),
            pltpu.SemaphoreType.DMA,
        ],
    )(_scatter_big_body)
    return f(ef, row)


def _scatter_small_body(sm_hbm, row_hbm, small_out, rowv, smv, zs, stable,
                        sem):
    cid = lax.axis_index("c")
    sid = lax.axis_index("s")
    wid = sid * NC + cid

    def zero_zs(e, carry):
        zs[e, pl.ds(0, 16)] = jnp.zeros((16,), _f32)
        return carry

    lax.fori_loop(0, ZCH, zero_zs, 0)
    for r in range(ROWS_PT // ZCH):
        start = sid * ROWS_PT + r * ZCH
        pltpu.sync_copy(zs, stable.at[pl.ds(start, ZCH)])
    plsc.subcore_barrier()

    base = wid * EPW

    def chunk_body(i, carry):
        off = base + i * CHUNK
        pltpu.sync_copy(row_hbm.at[pl.ds(off, CHUNK)], rowv)
        pltpu.sync_copy(sm_hbm.at[pl.ds(off, CHUNK)], smv)
        pltpu.sync_copy(smv, stable.at[rowv], add=True)
        return carry

    lax.fori_loop(0, NCHUNK, chunk_body, 0)
    plsc.subcore_barrier()

    start = sid * ROWS_PT
    pltpu.sync_copy(stable.at[pl.ds(start, ROWS_PT)],
                    small_out.at[cid, pl.ds(start, ROWS_PT)])


def _scatter_small(small, row):
    mesh = plsc.VectorSubcoreMesh(core_axis_name="c", subcore_axis_name="s")
    f = functools.partial(
        pl.kernel,
        out_type=jax.ShapeDtypeStruct((NC, N_TAB, 16), _f32),
        mesh=mesh,
        scratch_types=[
            pltpu.VMEM((CHUNK,), jnp.int32),
            pltpu.VMEM((CHUNK, 16), _f32),
            pltpu.VMEM((ZCH, 16), _f32),
            pltpu.VMEM_SHARED((N_TAB, 16), _f32),
            pltpu.SemaphoreType.DMA,
        ],
    )(_scatter_small_body)
    return f(small, row)


def _node_body(x_ref, crd_ref, pb_ref, ps_ref, wn1a, wn1b, bn1, wn2, bn2,
               xo, co):
    aggn = pb_ref[0] + pb_ref[1]
    sm = ps_ref[0] + ps_ref[1]
    cnt = jnp.clip(sm[:, 4:5], 1.0, None)
    aggc = sm[:, 0:4] / cnt
    co[...] = crd_ref[...] + aggc
    nh = _silu(jnp.dot(x_ref[...], wn1a[...], preferred_element_type=_f32)
               + jnp.dot(aggn, wn1b[...], preferred_element_type=_f32)
               + bn1[...])
    xo[...] = (x_ref[...] + jnp.dot(nh, wn2[...], preferred_element_type=_f32)
               + bn2[...])


def _node_mlp(x, crdp, part_big, part_small, wn1a, wn1b, bn1, wn2, bn2):
    blk = 1000
    full = lambda i: (0, 0)
    return pl.pallas_call(
        _node_body,
        grid=(N // blk,),
        in_specs=[
            pl.BlockSpec((blk, D), lambda i: (i, 0)),
            pl.BlockSpec((blk, 4), lambda i: (i, 0)),
            pl.BlockSpec((NC, blk, H), lambda i: (0, i, 0)),
            pl.BlockSpec((NC, blk, 16), lambda i: (0, i, 0)),
            pl.BlockSpec((D, H), full),
            pl.BlockSpec((H, H), full),
            pl.BlockSpec((1, H), full),
            pl.BlockSpec((H, D), full),
            pl.BlockSpec((1, D), full),
        ],
        out_specs=[
            pl.BlockSpec((blk, D), lambda i: (i, 0)),
            pl.BlockSpec((blk, 4), lambda i: (i, 0)),
        ],
        out_shape=[
            jax.ShapeDtypeStruct((N, D), _f32),
            jax.ShapeDtypeStruct((N, 4), _f32),
        ],
    )(x, crdp, part_big, part_small, wn1a, wn1b, bn1, wn2, bn2)


def kernel(x, edge_index, coord, edge_attr,
           W_e1, b_e1, W_e2, b_e2,
           W_n1, b_n1, W_n2, b_n2,
           W_c1, b_c1, W_c2):
    row = edge_index[0]
    col = edge_index[1]
    crdp = jnp.pad(coord, ((0, 0), (0, 1)))
    crd_flat = crdp.reshape(-1)

    A = W_e1[0:D]
    B = W_e1[D:2 * D]
    wr = W_e1[2 * D:2 * D + 1]
    cw = W_e1[2 * D + 1:]

    P, Q = _compute_pq(x, A, B)
    if _DEBUG_JNP_GATHER:
        gp, gq = P[row], Q[col]
        cdv = crdp[row] - crdp[col]
        cd = cdv.at[:, 3].set(jnp.sum(cdv * cdv, axis=1))
    else:
        gp, gq, cd = _gather(P, Q, crd_flat, row, col)
    ef, small = _edge_mlp(gp, gq, cd, edge_attr,
                          wr, cw, b_e1.reshape(1, H),
                          W_e2, b_e2.reshape(1, H),
                          W_c1, b_c1.reshape(1, H), W_c2)
    if _DEBUG_JNP_SCATTER:
        pb = jax.ops.segment_sum(ef, row, num_segments=N_TAB)[None]
        part_big = jnp.concatenate([pb, jnp.zeros_like(pb)], 0)
    else:
        part_big = _scatter_big(ef, row)
    if _DEBUG_JNP_SCATTER_SMALL:
        ps = jax.ops.segment_sum(small, row, num_segments=N_TAB)[None]
        part_small = jnp.concatenate([ps, jnp.zeros_like(ps)], 0)
    else:
        part_small = _scatter_small(small, row)
    x_new, crd_new = _node_mlp(x, crdp, part_big, part_small,
                               W_n1[:D], W_n1[D:], b_n1.reshape(1, H),
                               W_n2, b_n2.reshape(1, D))
    return (x_new, crd_new[:, :3], edge_attr)


# speedup vs baseline: 4.5989x; 1.4979x over previous
import functools

import jax
import jax.numpy as jnp
from jax import lax
from jax.experimental import pallas as pl
from jax.experimental.pallas import tpu as pltpu
from jax.experimental.pallas import tpu_sc as plsc

N = 10000
E = 320000
D = 128
H = 128
DE = 16

NC = 2
NS = 16
NW = NC * NS
EPW = E // NW
CHUNK = 80
NCHUNK = EPW // CHUNK
N_TAB = 10240
ROWS_PT = N_TAB // NS
ZCH = 128

_f32 = jnp.float32


def _silu(v):
    return v * (1.0 / (1.0 + jnp.exp(-v)))


def _pq_body(x_ref, a_ref, b_ref, p_ref, q_ref):
    xb = x_ref[...]
    p_ref[...] = jnp.dot(xb, a_ref[...], preferred_element_type=_f32)
    q_ref[...] = jnp.dot(xb, b_ref[...], preferred_element_type=_f32)


def _compute_pq(x, A, B):
    blk = 1000
    return pl.pallas_call(
        _pq_body,
        grid=(N // blk,),
        in_specs=[
            pl.BlockSpec((blk, D), lambda i: (i, 0)),
            pl.BlockSpec((D, H), lambda i: (0, 0)),
            pl.BlockSpec((D, H), lambda i: (0, 0)),
        ],
        out_specs=[pl.BlockSpec((blk, H), lambda i: (i, 0))] * 2,
        out_shape=[jax.ShapeDtypeStruct((N, H), _f32)] * 2,
    )(x, A, B)


CH1 = 160
SUB = 80
NCH1 = EPW // CH1
TAIL1 = EPW - NCH1 * CH1


def _gather_body(p_hbm, q_hbm, crd_hbm, row_hbm, col_hbm,
                 gp_out, gq_out, cd_out,
                 rowv, colv, pg, qg, cdb, crd_v, sem):
    pltpu.sync_copy(crd_hbm, crd_v)

    wid = lax.axis_index("s") * NC + lax.axis_index("c")
    base = wid * EPW

    def process(off, ch):
        i1 = pltpu.async_copy(row_hbm.at[pl.ds(off, ch)],
                              rowv.at[pl.ds(0, ch)], sem)
        i2 = pltpu.async_copy(col_hbm.at[pl.ds(off, ch)],
                              colv.at[pl.ds(0, ch)], sem)
        i1.wait()
        i2.wait()
        gs = []
        for k in range(ch // SUB):
            sl = pl.ds(k * SUB, SUB)
            gs.append(pltpu.async_copy(p_hbm.at[rowv.at[sl]], pg.at[sl], sem))
            gs.append(pltpu.async_copy(q_hbm.at[colv.at[sl]], qg.at[sl], sem))
        for g in range(ch // 16):
            rid = rowv[pl.ds(g * 16, 16)] * 3
            cid2 = colv[pl.ds(g * 16, 16)] * 3
            eidx = jnp.full((16,), g * 16, jnp.int32) + lax.iota(jnp.int32, 16)
            acc = jnp.zeros((16,), _f32)
            for k in range(3):
                a = plsc.load_gather(crd_v, [rid + k])
                b = plsc.load_gather(crd_v, [cid2 + k])
                dk = a - b
                acc = acc + dk * dk
                plsc.store_scatter(cdb, [eidx, jnp.full((16,), k, jnp.int32)],
                                   dk)
            plsc.store_scatter(cdb, [eidx, jnp.full((16,), 3, jnp.int32)],
                               acc)
        for c in gs:
            c.wait()
        sl = pl.ds(0, ch)
        w1 = pltpu.async_copy(pg.at[sl], gp_out.at[pl.ds(off, ch)], sem)
        w2 = pltpu.async_copy(qg.at[sl], gq_out.at[pl.ds(off, ch)], sem)
        w1.wait()
        w2.wait()
        pltpu.sync_copy(cdb.at[sl], cd_out.at[pl.ds(off, ch)])

    def chunk_body(i, carry):
        process(base + i * CH1, CH1)
        return carry

    lax.fori_loop(0, NCH1, chunk_body, 0)
    if TAIL1:
        process(base + NCH1 * CH1, TAIL1)


def _gather(P, Q, crd_flat, row, col):
    mesh = plsc.VectorSubcoreMesh(core_axis_name="c", subcore_axis_name="s")
    f = functools.partial(
        pl.kernel,
        out_type=(
            jax.ShapeDtypeStruct((E, H), _f32),
            jax.ShapeDtypeStruct((E, H), _f32),
            jax.ShapeDtypeStruct((E, 4), _f32),
        ),
        mesh=mesh,
        compiler_params=pltpu.CompilerParams(needs_layout_passes=False),
        scratch_types=[
            pltpu.VMEM((CH1,), jnp.int32),
            pltpu.VMEM((CH1,), jnp.int32),
            pltpu.VMEM((CH1, H), _f32),
            pltpu.VMEM((CH1, H), _f32),
            pltpu.VMEM((CH1, 4), _f32),
            pltpu.VMEM((3 * N,), _f32),
            pltpu.SemaphoreType.DMA,
        ],
    )(_gather_body)
    return f(P, Q, crd_flat, row, col)


def _edge_body(gp, gq, cd_ref, ea, wr, cw, be1, we2, be2, wc1, bc1, wc2,
               ef_o, sm_o):
    g = gp[...] + gq[...]
    cd = cd_ref[...]
    radial = cd[:, 3:4]
    pre = (g + radial * wr[...]
           + jnp.dot(ea[...], cw[...], preferred_element_type=_f32)
           + be1[...])
    h = _silu(pre)
    ef = _silu(jnp.dot(h, we2[...], preferred_element_type=_f32) + be2[...])
    ch = _silu(jnp.dot(ef, wc1[...], preferred_element_type=_f32) + bc1[...])
    s = jnp.dot(ch, wc2[...], preferred_element_type=_f32)
    ef_o[...] = ef
    blk = cd.shape[0]
    sm_o[...] = jnp.concatenate(
        [cd * s, jnp.ones((blk, 1), _f32), jnp.zeros((blk, H - 5), _f32)],
        axis=1)


def _edge_mlp(gp, gq, cd, edge_attr, wr, cw, be1, we2, be2, wc1, bc1, wc2):
    BE = 2000
    full = lambda i: (0, 0)
    return pl.pallas_call(
        _edge_body,
        grid=(E // BE,),
        in_specs=[
            pl.BlockSpec((BE, H), lambda i: (i, 0)),
            pl.BlockSpec((BE, H), lambda i: (i, 0)),
            pl.BlockSpec((BE, 4), lambda i: (i, 0)),
            pl.BlockSpec((BE, DE), lambda i: (i, 0)),
            pl.BlockSpec((1, H), full),
            pl.BlockSpec((DE, H), full),
            pl.BlockSpec((1, H), full),
            pl.BlockSpec((H, H), full),
            pl.BlockSpec((1, H), full),
            pl.BlockSpec((H, H), full),
            pl.BlockSpec((1, H), full),
            pl.BlockSpec((H, 1), full),
        ],
        out_specs=[
            pl.BlockSpec((BE, H), lambda i: (i, 0)),
            pl.BlockSpec((BE, H), lambda i: (i, 0)),
        ],
        out_shape=[
            jax.ShapeDtypeStruct((E, H), _f32),
            jax.ShapeDtypeStruct((E, H), _f32),
        ],
    )(gp, gq, cd, edge_attr, wr, cw, be1, we2, be2, wc1, bc1, wc2)


def _scatter_big_body(ef_hbm, row_hbm, big_out, rowv, efv, zb, table, sem):
    cid = lax.axis_index("c")
    sid = lax.axis_index("s")
    wid = sid * NC + cid

    def zero_zb(e, carry):
        for u in range(H // 16):
            zb[e, pl.ds(u * 16, 16)] = jnp.zeros((16,), _f32)
        return carry

    lax.fori_loop(0, ZCH, zero_zb, 0)
    for r in range(ROWS_PT // ZCH):
        start = sid * ROWS_PT + r * ZCH
        pltpu.sync_copy(zb, table.at[pl.ds(start, ZCH)])
    plsc.subcore_barrier()

    base = wid * EPW

    def chunk_body(i, carry):
        off = base + i * CHUNK
        c1 = pltpu.async_copy(row_hbm.at[pl.ds(off, CHUNK)], rowv, sem)
        c2 = pltpu.async_copy(ef_hbm.at[pl.ds(off, CHUNK)], efv, sem)
        c1.wait()
        c2.wait()
        pltpu.sync_copy(efv, table.at[rowv], add=True)
        return carry

    lax.fori_loop(0, NCHUNK, chunk_body, 0)
    plsc.subcore_barrier()

    start = sid * ROWS_PT
    pltpu.sync_copy(table.at[pl.ds(start, ROWS_PT)],
                    big_out.at[cid, pl.ds(start, ROWS_PT)])


def _scatter_big(ef, row):
    mesh = plsc.VectorSubcoreMesh(core_axis_name="c", subcore_axis_name="s")
    f = functools.partial(
        pl.kernel,
        out_type=jax.ShapeDtypeStruct((NC, N_TAB, H), _f32),
        mesh=mesh,
        scratch_types=[
            pltpu.VMEM((CHUNK,), jnp.int32),
            pltpu.VMEM((CHUNK, H), _f32),
            pltpu.VMEM((ZCH, H), _f32),
            pltpu.VMEM_SHARED((N_TAB, H), _f32),
            pltpu.SemaphoreType.DMA,
        ],
    )(_scatter_big_body)
    return f(ef, row)


def _node_body(x_ref, crd_ref, pb_ref, ps_ref, wn1a, wn1b, bn1, wn2, bn2,
               xo, co):
    aggn = pb_ref[0] + pb_ref[1]
    sm = ps_ref[0] + ps_ref[1]
    cnt = jnp.clip(sm[:, 4:5], 1.0, None)
    aggc = sm[:, 0:4] / cnt
    co[...] = crd_ref[...] + aggc
    nh = _silu(jnp.dot(x_ref[...], wn1a[...], preferred_element_type=_f32)
               + jnp.dot(aggn, wn1b[...], preferred_element_type=_f32)
               + bn1[...])
    xo[...] = (x_ref[...] + jnp.dot(nh, wn2[...], preferred_element_type=_f32)
               + bn2[...])


def _node_mlp(x, crdp, part_big, part_small, wn1a, wn1b, bn1, wn2, bn2):
    blk = 1000
    full = lambda i: (0, 0)
    return pl.pallas_call(
        _node_body,
        grid=(N // blk,),
        in_specs=[
            pl.BlockSpec((blk, D), lambda i: (i, 0)),
            pl.BlockSpec((blk, 4), lambda i: (i, 0)),
            pl.BlockSpec((NC, blk, H), lambda i: (0, i, 0)),
            pl.BlockSpec((NC, blk, H), lambda i: (0, i, 0)),
            pl.BlockSpec((D, H), full),
            pl.BlockSpec((H, H), full),
            pl.BlockSpec((1, H), full),
            pl.BlockSpec((H, D), full),
            pl.BlockSpec((1, D), full),
        ],
        out_specs=[
            pl.BlockSpec((blk, D), lambda i: (i, 0)),
            pl.BlockSpec((blk, 4), lambda i: (i, 0)),
        ],
        out_shape=[
            jax.ShapeDtypeStruct((N, D), _f32),
            jax.ShapeDtypeStruct((N, 4), _f32),
        ],
    )(x, crdp, part_big, part_small, wn1a, wn1b, bn1, wn2, bn2)


def kernel(x, edge_index, coord, edge_attr,
           W_e1, b_e1, W_e2, b_e2,
           W_n1, b_n1, W_n2, b_n2,
           W_c1, b_c1, W_c2):
    row = edge_index[0]
    col = edge_index[1]
    crdp = jnp.pad(coord, ((0, 0), (0, 1)))
    crd_flat = coord.reshape(-1)

    A = W_e1[0:D]
    B = W_e1[D:2 * D]
    wr = W_e1[2 * D:2 * D + 1]
    cw = W_e1[2 * D + 1:]

    P, Q = _compute_pq(x, A, B)
    gp, gq, cd = _gather(P, Q, crd_flat, row, col)
    ef, small = _edge_mlp(gp, gq, cd, edge_attr,
                          wr, cw, b_e1.reshape(1, H),
                          W_e2, b_e2.reshape(1, H),
                          W_c1, b_c1.reshape(1, H), W_c2)
    part_big = _scatter_big(ef, row)
    part_small = _scatter_big(small, row)
    x_new, crd_new = _node_mlp(x, crdp, part_big, part_small,
                               W_n1[:D], W_n1[D:], b_n1.reshape(1, H),
                               W_n2, b_n2.reshape(1, D))
    return (x_new, crd_new[:, :3], edge_attr)


# speedup vs baseline: 4.6006x; 1.0004x over previous
import functools

import jax
import jax.numpy as jnp
from jax import lax
from jax.experimental import pallas as pl
from jax.experimental.pallas import tpu as pltpu
from jax.experimental.pallas import tpu_sc as plsc

N = 10000
E = 320000
D = 128
H = 128
DE = 16

NC = 2
NS = 16
NW = NC * NS
EPW = E // NW
CHUNK = 80
NCHUNK = EPW // CHUNK
N_TAB = 10240
ROWS_PT = N_TAB // NS
ZCH = 128

_f32 = jnp.float32


def _silu(v):
    return v * (1.0 / (1.0 + jnp.exp(-v)))


def _pq_body(x_ref, a_ref, b_ref, p_ref, q_ref):
    xb = x_ref[...]
    p_ref[...] = jnp.dot(xb, a_ref[...], preferred_element_type=_f32)
    q_ref[...] = jnp.dot(xb, b_ref[...], preferred_element_type=_f32)


def _compute_pq(x, A, B):
    blk = 1000
    return pl.pallas_call(
        _pq_body,
        grid=(N // blk,),
        in_specs=[
            pl.BlockSpec((blk, D), lambda i: (i, 0)),
            pl.BlockSpec((D, H), lambda i: (0, 0)),
            pl.BlockSpec((D, H), lambda i: (0, 0)),
        ],
        out_specs=[pl.BlockSpec((blk, H), lambda i: (i, 0))] * 2,
        out_shape=[jax.ShapeDtypeStruct((N, H), _f32)] * 2,
    )(x, A, B)


CH1 = 160
SUB = 80
NCH1 = EPW // CH1
TAIL1 = EPW - NCH1 * CH1


def _gather_body(p_hbm, q_hbm, crd_hbm, row_hbm, col_hbm,
                 gp_out, gq_out, cd_out,
                 rowv, colv, pg, qg, cdb, crd_v, sem):
    pltpu.sync_copy(crd_hbm, crd_v)

    wid = lax.axis_index("s") * NC + lax.axis_index("c")
    base = wid * EPW

    def process(off, ch):
        i1 = pltpu.async_copy(row_hbm.at[pl.ds(off, ch)],
                              rowv.at[pl.ds(0, ch)], sem)
        i2 = pltpu.async_copy(col_hbm.at[pl.ds(off, ch)],
                              colv.at[pl.ds(0, ch)], sem)
        i1.wait()
        i2.wait()
        gs = []
        for k in range(ch // SUB):
            sl = pl.ds(k * SUB, SUB)
            gs.append(pltpu.async_copy(p_hbm.at[rowv.at[sl]], pg.at[sl], sem))
            gs.append(pltpu.async_copy(q_hbm.at[colv.at[sl]], qg.at[sl], sem))
        for g in range(ch // 16):
            rid = rowv[pl.ds(g * 16, 16)] * 3
            cid2 = colv[pl.ds(g * 16, 16)] * 3
            eidx = jnp.full((16,), g * 16, jnp.int32) + lax.iota(jnp.int32, 16)
            acc = jnp.zeros((16,), _f32)
            for k in range(3):
                a = plsc.load_gather(crd_v, [rid + k])
                b = plsc.load_gather(crd_v, [cid2 + k])
                dk = a - b
                acc = acc + dk * dk
                plsc.store_scatter(cdb, [eidx, jnp.full((16,), k, jnp.int32)],
                                   dk)
            plsc.store_scatter(cdb, [eidx, jnp.full((16,), 3, jnp.int32)],
                               acc)
        for c in gs:
            c.wait()
        sl = pl.ds(0, ch)
        w1 = pltpu.async_copy(pg.at[sl], gp_out.at[pl.ds(off, ch)], sem)
        w2 = pltpu.async_copy(qg.at[sl], gq_out.at[pl.ds(off, ch)], sem)
        w1.wait()
        w2.wait()
        pltpu.sync_copy(cdb.at[sl], cd_out.at[pl.ds(off, ch)])

    def chunk_body(i, carry):
        process(base + i * CH1, CH1)
        return carry

    lax.fori_loop(0, NCH1, chunk_body, 0)
    if TAIL1:
        process(base + NCH1 * CH1, TAIL1)


def _gather(P, Q, crd_flat, row, col):
    mesh = plsc.VectorSubcoreMesh(core_axis_name="c", subcore_axis_name="s")
    f = functools.partial(
        pl.kernel,
        out_type=(
            jax.ShapeDtypeStruct((E, H), _f32),
            jax.ShapeDtypeStruct((E, H), _f32),
            jax.ShapeDtypeStruct((E, 4), _f32),
        ),
        mesh=mesh,
        compiler_params=pltpu.CompilerParams(needs_layout_passes=False),
        scratch_types=[
            pltpu.VMEM((CH1,), jnp.int32),
            pltpu.VMEM((CH1,), jnp.int32),
            pltpu.VMEM((CH1, H), _f32),
            pltpu.VMEM((CH1, H), _f32),
            pltpu.VMEM((CH1, 4), _f32),
            pltpu.VMEM((3 * N,), _f32),
            pltpu.SemaphoreType.DMA,
        ],
    )(_gather_body)
    return f(P, Q, crd_flat, row, col)


def _edge_body(gp, gq, cd_ref, ea, wr, cw, be1, we2, be2, wc1, bc1, wc2,
               ef_o, sm_o):
    g = gp[...] + gq[...]
    cd = cd_ref[...]
    radial = cd[:, 3:4]
    pre = (g + radial * wr[...]
           + jnp.dot(ea[...], cw[...], preferred_element_type=_f32)
           + be1[...])
    h = _silu(pre)
    ef = _silu(jnp.dot(h, we2[...], preferred_element_type=_f32) + be2[...])
    ch = _silu(jnp.dot(ef, wc1[...], preferred_element_type=_f32) + bc1[...])
    s = jnp.dot(ch, wc2[...], preferred_element_type=_f32)
    ef_o[...] = ef
    blk = cd.shape[0]
    sm_o[...] = jnp.concatenate(
        [cd * s, jnp.ones((blk, 1), _f32), jnp.zeros((blk, H - 5), _f32)],
        axis=1)


def _edge_mlp(gp, gq, cd, edge_attr, wr, cw, be1, we2, be2, wc1, bc1, wc2):
    BE = 2000
    full = lambda i: (0, 0)
    return pl.pallas_call(
        _edge_body,
        grid=(E // BE,),
        in_specs=[
            pl.BlockSpec((BE, H), lambda i: (i, 0)),
            pl.BlockSpec((BE, H), lambda i: (i, 0)),
            pl.BlockSpec((BE, 4), lambda i: (i, 0)),
            pl.BlockSpec((BE, DE), lambda i: (i, 0)),
            pl.BlockSpec((1, H), full),
            pl.BlockSpec((DE, H), full),
            pl.BlockSpec((1, H), full),
            pl.BlockSpec((H, H), full),
            pl.BlockSpec((1, H), full),
            pl.BlockSpec((H, H), full),
            pl.BlockSpec((1, H), full),
            pl.BlockSpec((H, 1), full),
        ],
        out_specs=[
            pl.BlockSpec((BE, H), lambda i: (i, 0)),
            pl.BlockSpec((BE, H), lambda i: (i, 0)),
        ],
        out_shape=[
            jax.ShapeDtypeStruct((E, H), _f32),
            jax.ShapeDtypeStruct((E, H), _f32),
        ],
    )(gp, gq, cd, edge_attr, wr, cw, be1, we2, be2, wc1, bc1, wc2)


ZB = 64


def _make_scatter_body(width):
    def body(src_hbm, row_hbm, out, rowv, sv, zb, table, sem):
        cid = lax.axis_index("c")
        sid = lax.axis_index("s")
        wid = sid * NC + cid

        def zero_row(e, carry):
            for u in range(width // 16):
                zb[e, pl.ds(u * 16, 16)] = jnp.zeros((16,), _f32)
            return carry

        lax.fori_loop(0, ZB, zero_row, 0)
        for r in range(ROWS_PT // ZB):
            pltpu.sync_copy(zb, table.at[pl.ds(sid * ROWS_PT + r * ZB, ZB)])
        plsc.subcore_barrier()

        base = wid * EPW

        def chunk_body(i, carry):
            off = base + i * CHUNK
            c1 = pltpu.async_copy(row_hbm.at[pl.ds(off, CHUNK)], rowv, sem)
            c2 = pltpu.async_copy(src_hbm.at[pl.ds(off, CHUNK)], sv, sem)
            c1.wait()
            c2.wait()
            pltpu.sync_copy(sv, table.at[rowv], add=True)
            return carry

        lax.fori_loop(0, NCHUNK, chunk_body, 0)
        plsc.subcore_barrier()

        start = sid * ROWS_PT
        pltpu.sync_copy(table.at[pl.ds(start, ROWS_PT)],
                        out.at[cid, pl.ds(start, ROWS_PT)])

    return body


def _scatter(src, row, width):
    mesh = plsc.VectorSubcoreMesh(core_axis_name="c", subcore_axis_name="s")
    f = functools.partial(
        pl.kernel,
        out_type=jax.ShapeDtypeStruct((NC, N_TAB, width), _f32),
        mesh=mesh,
        scratch_types=[
            pltpu.VMEM((CHUNK,), jnp.int32),
            pltpu.VMEM((CHUNK, width), _f32),
            pltpu.VMEM((ZB, width), _f32),
            pltpu.VMEM_SHARED((N_TAB, width), _f32),
            pltpu.SemaphoreType.DMA,
        ],
    )(_make_scatter_body(width))
    return f(src, row)


def _node_body(x_ref, crd_ref, pb_ref, ps_ref, wn1a, wn1b, bn1, wn2, bn2,
               xo, co):
    aggn = pb_ref[0] + pb_ref[1]
    sm = ps_ref[0] + ps_ref[1]
    cnt = jnp.clip(sm[:, 4:5], 1.0, None)
    aggc = sm[:, 0:4] / cnt
    co[...] = crd_ref[...] + aggc
    nh = _silu(jnp.dot(x_ref[...], wn1a[...], preferred_element_type=_f32)
               + jnp.dot(aggn, wn1b[...], preferred_element_type=_f32)
               + bn1[...])
    xo[...] = (x_ref[...] + jnp.dot(nh, wn2[...], preferred_element_type=_f32)
               + bn2[...])


def _node_mlp(x, crdp, part_big, part_small, wn1a, wn1b, bn1, wn2, bn2):
    blk = 1000
    full = lambda i: (0, 0)
    return pl.pallas_call(
        _node_body,
        grid=(N // blk,),
        in_specs=[
            pl.BlockSpec((blk, D), lambda i: (i, 0)),
            pl.BlockSpec((blk, 4), lambda i: (i, 0)),
            pl.BlockSpec((NC, blk, H), lambda i: (0, i, 0)),
            pl.BlockSpec((NC, blk, H), lambda i: (0, i, 0)),
            pl.BlockSpec((D, H), full),
            pl.BlockSpec((H, H), full),
            pl.BlockSpec((1, H), full),
            pl.BlockSpec((H, D), full),
            pl.BlockSpec((1, D), full),
        ],
        out_specs=[
            pl.BlockSpec((blk, D), lambda i: (i, 0)),
            pl.BlockSpec((blk, 4), lambda i: (i, 0)),
        ],
        out_shape=[
            jax.ShapeDtypeStruct((N, D), _f32),
            jax.ShapeDtypeStruct((N, 4), _f32),
        ],
    )(x, crdp, part_big, part_small, wn1a, wn1b, bn1, wn2, bn2)


def kernel(x, edge_index, coord, edge_attr,
           W_e1, b_e1, W_e2, b_e2,
           W_n1, b_n1, W_n2, b_n2,
           W_c1, b_c1, W_c2):
    row = edge_index[0]
    col = edge_index[1]
    crdp = jnp.pad(coord, ((0, 0), (0, 1)))
    crd_flat = coord.reshape(-1)

    A = W_e1[0:D]
    B = W_e1[D:2 * D]
    wr = W_e1[2 * D:2 * D + 1]
    cw = W_e1[2 * D + 1:]

    P, Q = _compute_pq(x, A, B)
    gp, gq, cd = _gather(P, Q, crd_flat, row, col)
    ef, small = _edge_mlp(gp, gq, cd, edge_attr,
                          wr, cw, b_e1.reshape(1, H),
                          W_e2, b_e2.reshape(1, H),
                          W_c1, b_c1.reshape(1, H), W_c2)
    part_big = _scatter(ef, row, H)
    part_small = _scatter(small, row, H)
    x_new, crd_new = _node_mlp(x, crdp, part_big, part_small,
                               W_n1[:D], W_n1[D:], b_n1.reshape(1, H),
                               W_n2, b_n2.reshape(1, D))
    return (x_new, crd_new[:, :3], edge_attr)


# speedup vs baseline: 5.3900x; 1.1716x over previous
import functools

import jax
import jax.numpy as jnp
from jax import lax
from jax.experimental import pallas as pl
from jax.experimental.pallas import tpu as pltpu
from jax.experimental.pallas import tpu_sc as plsc

N = 10000
E = 320000
D = 128
H = 128
DE = 16

NC = 2
NS = 16
NW = NC * NS
EPW = E // NW
CHUNK = 80
NCHUNK = EPW // CHUNK
N_TAB = 10240
ROWS_PT = N_TAB // NS
ZCH = 128

_f32 = jnp.float32


def _silu(v):
    return v * (1.0 / (1.0 + jnp.exp(-v)))


def _pq_body(x_ref, a_ref, b_ref, p_ref, q_ref):
    xb = x_ref[...]
    p_ref[...] = jnp.dot(xb, a_ref[...], preferred_element_type=_f32)
    q_ref[...] = jnp.dot(xb, b_ref[...], preferred_element_type=_f32)


def _compute_pq(x, A, B):
    blk = 1000
    return pl.pallas_call(
        _pq_body,
        grid=(N // blk,),
        in_specs=[
            pl.BlockSpec((blk, D), lambda i: (i, 0)),
            pl.BlockSpec((D, H), lambda i: (0, 0)),
            pl.BlockSpec((D, H), lambda i: (0, 0)),
        ],
        out_specs=[pl.BlockSpec((blk, H), lambda i: (i, 0))] * 2,
        out_shape=[jax.ShapeDtypeStruct((N, H), _f32)] * 2,
    )(x, A, B)


CH1 = 160
SUB = 80
NCH1 = EPW // CH1
TAIL1 = EPW - NCH1 * CH1


NPAIR1 = NCH1 // 2 - 1


def _gather_body(p_hbm, q_hbm, crd_hbm, row_hbm, col_hbm,
                 gp_out, gq_out, cd_out,
                 rowva, colva, rowvb, colvb, pg, qg, cdb, crd_v, semi, sem):
    pltpu.sync_copy(crd_hbm, crd_v)

    wid = lax.axis_index("s") * NC + lax.axis_index("c")
    base = wid * EPW

    def issue_idx(off, rv, cv, ch):
        pltpu.async_copy(row_hbm.at[pl.ds(off, ch)], rv.at[pl.ds(0, ch)],
                         semi)
        pltpu.async_copy(col_hbm.at[pl.ds(off, ch)], cv.at[pl.ds(0, ch)],
                         semi)

    def wait_idx(off, rv, cv, ch):
        pltpu.make_async_copy(row_hbm.at[pl.ds(off, ch)],
                              rv.at[pl.ds(0, ch)], semi).wait()
        pltpu.make_async_copy(col_hbm.at[pl.ds(off, ch)],
                              cv.at[pl.ds(0, ch)], semi).wait()

    def process(off, rv, cv, ch):
        gs = []
        for k in range(ch // SUB):
            sl = pl.ds(k * SUB, SUB)
            gs.append(pltpu.async_copy(p_hbm.at[rv.at[sl]], pg.at[sl], sem))
            gs.append(pltpu.async_copy(q_hbm.at[cv.at[sl]], qg.at[sl], sem))
        for g in range(ch // 16):
            rid = rv[pl.ds(g * 16, 16)] * 3
            cid2 = cv[pl.ds(g * 16, 16)] * 3
            eidx = jnp.full((16,), g * 16, jnp.int32) + lax.iota(jnp.int32, 16)
            acc = jnp.zeros((16,), _f32)
            for k in range(3):
                a = plsc.load_gather(crd_v, [rid + k])
                bb = plsc.load_gather(crd_v, [cid2 + k])
                dk = a - bb
                acc = acc + dk * dk
                plsc.store_scatter(cdb, [eidx, jnp.full((16,), k, jnp.int32)],
                                   dk)
            plsc.store_scatter(cdb, [eidx, jnp.full((16,), 3, jnp.int32)],
                               acc)
        for c in gs:
            c.wait()
        sl = pl.ds(0, ch)
        w1 = pltpu.async_copy(pg.at[sl], gp_out.at[pl.ds(off, ch)], sem)
        w2 = pltpu.async_copy(qg.at[sl], gq_out.at[pl.ds(off, ch)], sem)
        w1.wait()
        w2.wait()
        pltpu.sync_copy(cdb.at[sl], cd_out.at[pl.ds(off, ch)])

    issue_idx(base, rowva, colva, CH1)

    def pair_body(j, carry):
        offa = base + (2 * j) * CH1
        issue_idx(offa + CH1, rowvb, colvb, CH1)
        wait_idx(offa, rowva, colva, CH1)
        process(offa, rowva, colva, CH1)
        issue_idx(offa + 2 * CH1, rowva, colva, CH1)
        wait_idx(offa + CH1, rowvb, colvb, CH1)
        process(offa + CH1, rowvb, colvb, CH1)
        return carry

    lax.fori_loop(0, NPAIR1, pair_body, 0)
    offa = base + (NCH1 - 2) * CH1
    issue_idx(offa + CH1, rowvb, colvb, CH1)
    wait_idx(offa, rowva, colva, CH1)
    process(offa, rowva, colva, CH1)
    wait_idx(offa + CH1, rowvb, colvb, CH1)
    process(offa + CH1, rowvb, colvb, CH1)
    if TAIL1:
        toff = base + NCH1 * CH1
        issue_idx(toff, rowva, colva, TAIL1)
        wait_idx(toff, rowva, colva, TAIL1)
        process(toff, rowva, colva, TAIL1)


def _gather(P, Q, crd_flat, row, col):
    mesh = plsc.VectorSubcoreMesh(core_axis_name="c", subcore_axis_name="s")
    f = functools.partial(
        pl.kernel,
        out_type=(
            jax.ShapeDtypeStruct((E, H), _f32),
            jax.ShapeDtypeStruct((E, H), _f32),
            jax.ShapeDtypeStruct((E, 4), _f32),
        ),
        mesh=mesh,
        compiler_params=pltpu.CompilerParams(needs_layout_passes=False),
        scratch_types=[
            pltpu.VMEM((CH1,), jnp.int32),
            pltpu.VMEM((CH1,), jnp.int32),
            pltpu.VMEM((CH1,), jnp.int32),
            pltpu.VMEM((CH1,), jnp.int32),
            pltpu.VMEM((CH1, H), _f32),
            pltpu.VMEM((CH1, H), _f32),
            pltpu.VMEM((CH1, 4), _f32),
            pltpu.VMEM((3 * N,), _f32),
            pltpu.SemaphoreType.DMA,
            pltpu.SemaphoreType.DMA,
        ],
    )(_gather_body)
    return f(P, Q, crd_flat, row, col)


def _edge_body(gp, gq, cd_ref, ea, wr, cw, be1, we2, be2, wc1, bc1, wc2,
               ef_o, sm_o):
    g = gp[...] + gq[...]
    cd = cd_ref[...]
    radial = cd[:, 3:4]
    pre = (g + radial * wr[...]
           + jnp.dot(ea[...], cw[...], preferred_element_type=_f32)
           + be1[...])
    h = _silu(pre)
    ef = _silu(jnp.dot(h, we2[...], preferred_element_type=_f32) + be2[...])
    ch = _silu(jnp.dot(ef, wc1[...], preferred_element_type=_f32) + bc1[...])
    s = jnp.dot(ch, wc2[...], preferred_element_type=_f32)
    ef_o[...] = ef
    blk = cd.shape[0]
    sm_o[...] = jnp.concatenate(
        [cd * s, jnp.ones((blk, 1), _f32), jnp.zeros((blk, H - 5), _f32)],
        axis=1)


def _edge_mlp(gp, gq, cd, edge_attr, wr, cw, be1, we2, be2, wc1, bc1, wc2):
    BE = 2000
    full = lambda i: (0, 0)
    return pl.pallas_call(
        _edge_body,
        grid=(E // BE,),
        in_specs=[
            pl.BlockSpec((BE, H), lambda i: (i, 0)),
            pl.BlockSpec((BE, H), lambda i: (i, 0)),
            pl.BlockSpec((BE, 4), lambda i: (i, 0)),
            pl.BlockSpec((BE, DE), lambda i: (i, 0)),
            pl.BlockSpec((1, H), full),
            pl.BlockSpec((DE, H), full),
            pl.BlockSpec((1, H), full),
            pl.BlockSpec((H, H), full),
            pl.BlockSpec((1, H), full),
            pl.BlockSpec((H, H), full),
            pl.BlockSpec((1, H), full),
            pl.BlockSpec((H, 1), full),
        ],
        out_specs=[
            pl.BlockSpec((BE, H), lambda i: (i, 0)),
            pl.BlockSpec((BE, H), lambda i: (i, 0)),
        ],
        out_shape=[
            jax.ShapeDtypeStruct((E, H), _f32),
            jax.ShapeDtypeStruct((E, H), _f32),
        ],
    )(gp, gq, cd, edge_attr, wr, cw, be1, we2, be2, wc1, bc1, wc2)


ZB = 64


NPAIRS = NCHUNK // 2


def _make_scatter_body(width):
    def body(src_hbm, row_hbm, out, rowva, sva, rowvb, svb, zb, table, sem):
        cid = lax.axis_index("c")
        sid = lax.axis_index("s")
        wid = sid * NC + cid

        def zero_row(e, carry):
            for u in range(width // 16):
                zb[e, pl.ds(u * 16, 16)] = jnp.zeros((16,), _f32)
            return carry

        lax.fori_loop(0, ZB, zero_row, 0)
        for r in range(ROWS_PT // ZB):
            pltpu.sync_copy(zb, table.at[pl.ds(sid * ROWS_PT + r * ZB, ZB)])
        plsc.subcore_barrier()

        base = wid * EPW

        def issue(off, rv, pv):
            pltpu.async_copy(row_hbm.at[pl.ds(off, CHUNK)], rv, sem)
            pltpu.async_copy(src_hbm.at[pl.ds(off, CHUNK)], pv, sem)

        def wait_scatter(off, rv, pv):
            pltpu.make_async_copy(
                row_hbm.at[pl.ds(off, CHUNK)], rv, sem).wait()
            pltpu.make_async_copy(
                src_hbm.at[pl.ds(off, CHUNK)], pv, sem).wait()
            pltpu.sync_copy(pv, table.at[rv], add=True)

        issue(base, rowva, sva)

        def pair_body(j, carry):
            offa = base + (2 * j) * CHUNK
            issue(offa + CHUNK, rowvb, svb)
            wait_scatter(offa, rowva, sva)
            issue(offa + 2 * CHUNK, rowva, sva)
            wait_scatter(offa + CHUNK, rowvb, svb)
            return carry

        lax.fori_loop(0, NPAIRS, pair_body, 0)
        wait_scatter(base + (NCHUNK - 1) * CHUNK, rowva, sva)
        plsc.subcore_barrier()

        start = sid * ROWS_PT
        pltpu.sync_copy(table.at[pl.ds(start, ROWS_PT)],
                        out.at[cid, pl.ds(start, ROWS_PT)])

    return body


def _scatter(src, row, width):
    mesh = plsc.VectorSubcoreMesh(core_axis_name="c", subcore_axis_name="s")
    f = functools.partial(
        pl.kernel,
        out_type=jax.ShapeDtypeStruct((NC, N_TAB, width), _f32),
        mesh=mesh,
        scratch_types=[
            pltpu.VMEM((CHUNK,), jnp.int32),
            pltpu.VMEM((CHUNK, width), _f32),
            pltpu.VMEM((CHUNK,), jnp.int32),
            pltpu.VMEM((CHUNK, width), _f32),
            pltpu.VMEM((ZB, width), _f32),
            pltpu.VMEM_SHARED((N_TAB, width), _f32),
            pltpu.SemaphoreType.DMA,
        ],
    )(_make_scatter_body(width))
    return f(src, row)


def _node_body(x_ref, crd_ref, pb_ref, ps_ref, wn1a, wn1b, bn1, wn2, bn2,
               xo, co):
    aggn = pb_ref[0] + pb_ref[1]
    sm = ps_ref[0] + ps_ref[1]
    cnt = jnp.clip(sm[:, 4:5], 1.0, None)
    aggc = sm[:, 0:4] / cnt
    co[...] = crd_ref[...] + aggc
    nh = _silu(jnp.dot(x_ref[...], wn1a[...], preferred_element_type=_f32)
               + jnp.dot(aggn, wn1b[...], preferred_element_type=_f32)
               + bn1[...])
    xo[...] = (x_ref[...] + jnp.dot(nh, wn2[...], preferred_element_type=_f32)
               + bn2[...])


def _node_mlp(x, crdp, part_big, part_small, wn1a, wn1b, bn1, wn2, bn2):
    blk = 1000
    full = lambda i: (0, 0)
    return pl.pallas_call(
        _node_body,
        grid=(N // blk,),
        in_specs=[
            pl.BlockSpec((blk, D), lambda i: (i, 0)),
            pl.BlockSpec((blk, 4), lambda i: (i, 0)),
            pl.BlockSpec((NC, blk, H), lambda i: (0, i, 0)),
            pl.BlockSpec((NC, blk, H), lambda i: (0, i, 0)),
            pl.BlockSpec((D, H), full),
            pl.BlockSpec((H, H), full),
            pl.BlockSpec((1, H), full),
            pl.BlockSpec((H, D), full),
            pl.BlockSpec((1, D), full),
        ],
        out_specs=[
            pl.BlockSpec((blk, D), lambda i: (i, 0)),
            pl.BlockSpec((blk, 4), lambda i: (i, 0)),
        ],
        out_shape=[
            jax.ShapeDtypeStruct((N, D), _f32),
            jax.ShapeDtypeStruct((N, 4), _f32),
        ],
    )(x, crdp, part_big, part_small, wn1a, wn1b, bn1, wn2, bn2)


def kernel(x, edge_index, coord, edge_attr,
           W_e1, b_e1, W_e2, b_e2,
           W_n1, b_n1, W_n2, b_n2,
           W_c1, b_c1, W_c2):
    row = edge_index[0]
    col = edge_index[1]
    crdp = jnp.pad(coord, ((0, 0), (0, 1)))
    crd_flat = coord.reshape(-1)

    A = W_e1[0:D]
    B = W_e1[D:2 * D]
    wr = W_e1[2 * D:2 * D + 1]
    cw = W_e1[2 * D + 1:]

    P, Q = _compute_pq(x, A, B)
    gp, gq, cd = _gather(P, Q, crd_flat, row, col)
    ef, small = _edge_mlp(gp, gq, cd, edge_attr,
                          wr, cw, b_e1.reshape(1, H),
                          W_e2, b_e2.reshape(1, H),
                          W_c1, b_c1.reshape(1, H), W_c2)
    part_big = _scatter(ef, row, H)
    part_small = _scatter(small, row, H)
    x_new, crd_new = _node_mlp(x, crdp, part_big, part_small,
                               W_n1[:D], W_n1[D:], b_n1.reshape(1, H),
                               W_n2, b_n2.reshape(1, D))
    return (x_new, crd_new[:, :3], edge_attr)
